# Initial kernel scaffold; baseline (speedup 1.0000x reference)
import jax
import jax.numpy as jnp
from jax.experimental import pallas as pl


def kernel(x, edge_index, batch, params):
    raise NotImplementedError("write your pallas kernel here")


# speedup vs baseline: 3.1986x; 3.1986x over previous
import functools

import jax
import jax.numpy as jnp
from jax import lax
from jax.experimental import pallas as pl
from jax.experimental.pallas import tpu as pltpu
from jax.experimental.pallas import tpu_sc as plsc

F32 = jnp.float32
N = 10000
E = 160000
H = 4
C = 256
HC = H * C

NC, NS = 2, 16
NW = NC * NS
EPW = E // NW
EPT3 = E // NS
B1 = 40
B2 = 40
B3 = 80
NPT = 320
NG = NW * NPT
DB = 2000
NR = E // DB
GC = 16


def _mm(x, w, b=None, act=None, rows=400):
    n, k = x.shape
    m = w.shape[1]
    bb = jnp.zeros((1, m), F32) if b is None else b.reshape(1, m)

    def body(x_ref, w_ref, b_ref, o_ref):
        acc = jnp.dot(x_ref[...], w_ref[...], preferred_element_type=F32,
                      precision=lax.Precision.HIGHEST)
        acc = acc + b_ref[...]
        if act is not None:
            acc = act(acc)
        o_ref[...] = acc

    return pl.pallas_call(
        body,
        grid=(n // rows,),
        in_specs=[pl.BlockSpec((rows, k), lambda i: (i, 0)),
                  pl.BlockSpec((k, m), lambda i: (0, 0)),
                  pl.BlockSpec((1, m), lambda i: (0, 0))],
        out_specs=pl.BlockSpec((rows, m), lambda i: (i, 0)),
        out_shape=jax.ShapeDtypeStruct((n, m), F32),
    )(x, w, bb)


def _post_a(hprev, agg, cb):
    rows = 1000
    g = N // rows

    def body(hp_ref, agg_ref, cb_ref, t_ref, ps_ref):
        t = jnp.maximum(hp_ref[...] + agg_ref[...] * (1.0 / H) + cb_ref[...],
                        0.0)
        t_ref[...] = t
        r8 = t.reshape(rows // 8, 8, C)
        ps_ref[0, 0] = jnp.sum(r8, axis=0)
        ps_ref[0, 1] = jnp.sum(r8 * r8, axis=0)

    return pl.pallas_call(
        body,
        grid=(g,),
        in_specs=[pl.BlockSpec((rows, C), lambda i: (i, 0)),
                  pl.BlockSpec((rows, C), lambda i: (i, 0)),
                  pl.BlockSpec((1, C), lambda i: (0, 0))],
        out_specs=[pl.BlockSpec((rows, C), lambda i: (i, 0)),
                   pl.BlockSpec((1, 2, 8, C), lambda i: (i, 0, 0, 0))],
        out_shape=[jax.ShapeDtypeStruct((N, C), F32),
                   jax.ShapeDtypeStruct((g, 2, 8, C), F32)],
    )(hprev, agg, cb.reshape(1, C))


def _post_b(t, ps, nw, nb, nms):
    rows = 1000
    g = N // rows

    def body(t_ref, ps_ref, w_ref, b_ref, ms_ref, o_ref):
        ps = ps_ref[...]
        mean = jnp.sum(ps[:, 0], axis=(0, 1)) * (1.0 / N)
        msq = jnp.sum(ps[:, 1], axis=(0, 1)) * (1.0 / N)
        mm = mean * ms_ref[0]
        var = msq - mm * (2.0 * mean - mm)
        tt = t_ref[...]
        o_ref[...] = (tt - mm) * lax.rsqrt(var + 1e-5) * w_ref[0] + b_ref[0]

    return pl.pallas_call(
        body,
        grid=(g,),
        in_specs=[pl.BlockSpec((rows, C), lambda i: (i, 0)),
                  pl.BlockSpec((g, 2, 8, C), lambda i: (0, 0, 0, 0)),
                  pl.BlockSpec((1, C), lambda i: (0, 0)),
                  pl.BlockSpec((1, C), lambda i: (0, 0)),
                  pl.BlockSpec((1, C), lambda i: (0, 0))],
        out_specs=pl.BlockSpec((rows, C), lambda i: (i, 0)),
        out_shape=jax.ShapeDtypeStruct((N, C), F32),
    )(t, ps, nw.reshape(1, C), nb.reshape(1, C), nms.reshape(1, C))


def _head(h, w1, b1, s1, t1, w2, b2, s2, t2, pw, pb):
    rows = 400
    g = N // rows

    def body(h_ref, w1_ref, b1_ref, s1_ref, t1_ref, w2_ref, b2_ref,
             s2_ref, t2_ref, pw_ref, pb_ref, ho_ref, po_ref):
        a = jnp.dot(h_ref[...], w1_ref[...], preferred_element_type=F32,
                    precision=lax.Precision.HIGHEST) + b1_ref[...]
        a = jnp.maximum(a, 0.0)
        a = a * s1_ref[...] + t1_ref[...]
        h2 = jnp.dot(a, w2_ref[...], preferred_element_type=F32,
                     precision=lax.Precision.HIGHEST) + b2_ref[...]
        ho_ref[...] = h2
        pin = h2 * s2_ref[...] + t2_ref[...]
        po_ref[...] = jnp.dot(pin, pw_ref[...], preferred_element_type=F32,
                              precision=lax.Precision.HIGHEST) + pb_ref[...]

    vec = lambda v: v.reshape(1, -1)
    return pl.pallas_call(
        body,
        grid=(g,),
        in_specs=[pl.BlockSpec((rows, C), lambda i: (i, 0)),
                  pl.BlockSpec((C, C), lambda i: (0, 0)),
                  pl.BlockSpec((1, C), lambda i: (0, 0)),
                  pl.BlockSpec((1, C), lambda i: (0, 0)),
                  pl.BlockSpec((1, C), lambda i: (0, 0)),
                  pl.BlockSpec((C, C), lambda i: (0, 0)),
                  pl.BlockSpec((1, C), lambda i: (0, 0)),
                  pl.BlockSpec((1, C), lambda i: (0, 0)),
                  pl.BlockSpec((1, C), lambda i: (0, 0)),
                  pl.BlockSpec((C, 128), lambda i: (0, 0)),
                  pl.BlockSpec((1, 128), lambda i: (0, 0))],
        out_specs=[pl.BlockSpec((rows, C), lambda i: (i, 0)),
                   pl.BlockSpec((rows, 128), lambda i: (i, 0))],
        out_shape=[jax.ShapeDtypeStruct((N, C), F32),
                   jax.ShapeDtypeStruct((N, 128), F32)],
    )(h, w1, vec(b1), vec(s1), vec(t1), w2, vec(b2), vec(s2), vec(t2),
      pw, vec(pb))


def _sc_mesh():
    return plsc.VectorSubcoreMesh(core_axis_name="c", subcore_axis_name="s")


def _permute(v, idx):
    return lax.gather(
        v, idx[:, None],
        lax.GatherDimensionNumbers(offset_dims=(), collapsed_slice_dims=(0,),
                                   start_index_map=(0,)),
        (1,), mode=lax.GatherScatterMode.PROMISE_IN_BOUNDS)


def _compact_chunk(dstb, eidb, lidb, kk, r, nodebase, cnt, ii16):
    dv = dstb[pl.ds(kk * 16, 16)]
    loc = dv - nodebase
    m = (loc >= 0) & (loc < NPT)
    eid = r * DB + kk * 16 + ii16
    mi = jnp.where(m, 1, 0).astype(jnp.int32)
    s = mi
    for d in (1, 2, 4, 8):
        sh = _permute(s, jnp.maximum(ii16 - d, 0))
        s = s + jnp.where(ii16 >= d, sh, 0)
    tgt = jnp.where(m, cnt + s - mi, DB + GC)
    plsc.store_scatter(eidb, [tgt], eid)
    plsc.store_scatter(lidb, [tgt], loc)
    return cnt + s[15]


def _p1(xl, xr, src, dst, attf):
    @functools.partial(
        pl.kernel,
        out_type=jax.ShapeDtypeStruct((E, 128), F32),
        mesh=_sc_mesh(),
        compiler_params=pltpu.CompilerParams(needs_layout_passes=False),
        scratch_types=[pltpu.VMEM((B1,), jnp.int32),
                       pltpu.VMEM((B1,), jnp.int32),
                       pltpu.VMEM((B1, HC), F32),
                       pltpu.VMEM((B1, HC), F32),
                       pltpu.VMEM((B1, 128), F32),
                       pltpu.VMEM((HC,), F32),
                       pltpu.SemaphoreType.DMA,
                       pltpu.SemaphoreType.DMA],
    )
    def k(xl_h, xr_h, src_h, dst_h, att_h, ex_h,
          isrc, idst, rl, rr, exb, attv, sem1, sem2):
        cid = lax.axis_index("c")
        sid = lax.axis_index("s")
        wid = sid * NC + cid

        pltpu.sync_copy(att_h, attv)
        z16 = jnp.zeros((16,), F32)

        def zrow(i, _):
            for kk in range(1, 8):
                exb[i, pl.ds(kk * 16, 16)] = z16
            return 0

        lax.fori_loop(0, B1, zrow, 0)
        base = wid * EPW
        ii16 = lax.broadcasted_iota(jnp.int32, (16,), 0)

        def batch(j, _):
            off = base + j * B1
            pltpu.sync_copy(src_h.at[pl.ds(off, B1)], isrc)
            pltpu.sync_copy(dst_h.at[pl.ds(off, B1)], idst)
            cp1 = pltpu.async_copy(xl_h.at[isrc], rl, sem1)
            cp2 = pltpu.async_copy(xr_h.at[idst], rr, sem2)
            cp1.wait()
            cp2.wait()

            def edge(i, _):
                lv = jnp.full((16,), -1e30, F32)
                for h in range(H):
                    def chunk(kk, acc, h=h):
                        o = h * C + kk * 16
                        u = rl[i, pl.ds(o, 16)] + rr[i, pl.ds(o, 16)]
                        return acc + jnp.maximum(u, 0.2 * u) * attv[pl.ds(o, 16)]
                    a = lax.fori_loop(0, C // 16, chunk, z16)
                    sh = a[0]
                    for l in range(1, 16):
                        sh = sh + a[l]
                    lv = jnp.where(ii16 == h, sh, lv)
                exb[i, pl.ds(0, 16)] = jnp.exp(lv)
                return 0

            lax.fori_loop(0, B1, edge, 0)
            pltpu.sync_copy(exb, ex_h.at[pl.ds(off, B1)])
            return 0

        lax.fori_loop(0, EPW // B1, batch, 0)

    return k(xl, xr, src, dst, attf)


def _p1b(ex, dst):
    @functools.partial(
        pl.kernel,
        out_type=jax.ShapeDtypeStruct((NG, 128), F32),
        mesh=_sc_mesh(),
        compiler_params=pltpu.CompilerParams(needs_layout_passes=False),
        scratch_types=[pltpu.VMEM((DB,), jnp.int32),
                       pltpu.VMEM((DB + 48,), jnp.int32),
                       pltpu.VMEM((DB + 48,), jnp.int32),
                       pltpu.VMEM((GC, 128), F32),
                       pltpu.VMEM((NPT + 8, 16), F32),
                       pltpu.VMEM((NPT, 128), F32),
                       pltpu.SemaphoreType.DMA],
    )
    def k(ex_h, dst_h, dfull_h, dstb, eidb, lidb, rowsb, acc, stage, sem):
        cid = lax.axis_index("c")
        sid = lax.axis_index("s")
        g = sid * NC + cid
        nodebase = g * NPT

        z16 = jnp.zeros((16,), F32)
        zi16 = jnp.zeros((16,), jnp.int32)
        ti16 = jnp.full((16,), NPT, jnp.int32)
        ii16 = lax.broadcasted_iota(jnp.int32, (16,), 0)

        def zacc(r, _):
            acc[r, :] = z16
            return 0

        lax.fori_loop(0, NPT + 8, zacc, 0)

        def zstage(r, _):
            for kk in range(8):
                stage[r, pl.ds(kk * 16, 16)] = z16
            return 0

        lax.fori_loop(0, NPT, zstage, 0)

        def rnd(r, _):
            pltpu.sync_copy(dst_h.at[pl.ds(r * DB, DB)], dstb)
            cnt = jnp.int32(0)
            for kk in range(DB // 16):
                cnt = _compact_chunk(dstb, eidb, lidb, kk, r, nodebase,
                                     cnt, ii16)
            eidb[pl.ds(cnt, 16)] = zi16
            lidb[pl.ds(cnt, 16)] = ti16
            ngrp = (cnt + (GC - 1)) // GC

            def grp(t, _):
                o = t * GC
                pltpu.async_copy(ex_h.at[eidb.at[pl.ds(o, GC)]], rowsb,
                                 sem).wait()
                lix = lidb[pl.ds(o, GC)]

                def chan(c_, _):
                    cc = jnp.full((16,), c_, jnp.int32)
                    v = plsc.load_gather(rowsb, [ii16, cc])
                    plsc.addupdate_scatter(acc, [lix, cc], v)
                    return 0

                lax.fori_loop(0, 16, chan, 0)
                return 0

            lax.fori_loop(0, ngrp, grp, 0)
            return 0

        lax.fori_loop(0, NR, rnd, 0)

        def srow(i, _):
            stage[i, pl.ds(0, 16)] = acc[i, :]
            return 0

        lax.fori_loop(0, NPT, srow, 0)
        pltpu.sync_copy(stage, dfull_h.at[pl.ds(nodebase, NPT)])

    return k(ex, dst)


def _p2(xl, ex, dfull, src, dst):
    @functools.partial(
        pl.kernel,
        out_type=jax.ShapeDtypeStruct((E, C), F32),
        mesh=_sc_mesh(),
        compiler_params=pltpu.CompilerParams(needs_layout_passes=False),
        scratch_types=[pltpu.VMEM((B2,), jnp.int32),
                       pltpu.VMEM((B2,), jnp.int32),
                       pltpu.VMEM((B2, HC), F32),
                       pltpu.VMEM((B2, 128), F32),
                       pltpu.VMEM((B2, 128), F32),
                       pltpu.VMEM((B2, C), F32),
                       pltpu.SemaphoreType.DMA,
                       pltpu.SemaphoreType.DMA],
    )
    def k(xl_h, ex_h, dfull_h, src_h, dst_h, msg_h,
          isrc, idst, rl, exb, db, msgb, sem1, sem2):
        cid = lax.axis_index("c")
        sid = lax.axis_index("s")
        wid = sid * NC + cid
        base = wid * EPW

        def batch(j, _):
            off = base + j * B2
            pltpu.sync_copy(src_h.at[pl.ds(off, B2)], isrc)
            pltpu.sync_copy(dst_h.at[pl.ds(off, B2)], idst)
            cp1 = pltpu.async_copy(xl_h.at[isrc], rl, sem1)
            cp2 = pltpu.async_copy(dfull_h.at[idst], db, sem2)
            pltpu.sync_copy(ex_h.at[pl.ds(off, B2)], exb)
            cp1.wait()
            cp2.wait()

            def edge(i, _):
                av = exb[i, pl.ds(0, 16)] / (db[i, pl.ds(0, 16)] + 1e-16)
                aa = [lax.squeeze(lax.slice(av, (h,), (h + 1,)), (0,))
                      for h in range(H)]
                for kk in range(C // 16):
                    mv = aa[0] * rl[i, pl.ds(kk * 16, 16)]
                    mv = mv + aa[1] * rl[i, pl.ds(C + kk * 16, 16)]
                    mv = mv + aa[2] * rl[i, pl.ds(2 * C + kk * 16, 16)]
                    mv = mv + aa[3] * rl[i, pl.ds(3 * C + kk * 16, 16)]
                    msgb[i, pl.ds(kk * 16, 16)] = mv
                return 0

            lax.fori_loop(0, B2, edge, 0)
            pltpu.sync_copy(msgb, msg_h.at[pl.ds(off, B2)])
            return 0

        lax.fori_loop(0, EPW // B2, batch, 0)

    return k(xl, ex, dfull, src, dst)


def _p3(msg, dst):
    @functools.partial(
        pl.kernel,
        out_type=jax.ShapeDtypeStruct((NG, C), F32),
        mesh=_sc_mesh(),
        compiler_params=pltpu.CompilerParams(needs_layout_passes=False),
        scratch_types=[pltpu.VMEM((DB,), jnp.int32),
                       pltpu.VMEM((DB + 48,), jnp.int32),
                       pltpu.VMEM((DB + 48,), jnp.int32),
                       pltpu.VMEM((GC, C), F32),
                       pltpu.VMEM((NPT + 8, C), F32),
                       pltpu.SemaphoreType.DMA],
    )
    def k(msg_h, dst_h, agg_h, dstb, eidb, lidb, rowsb, acc, sem):
        cid = lax.axis_index("c")
        sid = lax.axis_index("s")
        g = sid * NC + cid
        nodebase = g * NPT

        z16 = jnp.zeros((16,), F32)
        zi16 = jnp.zeros((16,), jnp.int32)
        ti16 = jnp.full((16,), NPT, jnp.int32)
        ii16 = lax.broadcasted_iota(jnp.int32, (16,), 0)

        def zacc(r, _):
            for kk in range(C // 16):
                acc[r, pl.ds(kk * 16, 16)] = z16
            return 0

        lax.fori_loop(0, NPT + 8, zacc, 0)

        def rnd(r, _):
            pltpu.sync_copy(dst_h.at[pl.ds(r * DB, DB)], dstb)
            cnt = jnp.int32(0)
            for kk in range(DB // 16):
                cnt = _compact_chunk(dstb, eidb, lidb, kk, r, nodebase,
                                     cnt, ii16)
            eidb[pl.ds(cnt, 16)] = zi16
            lidb[pl.ds(cnt, 16)] = ti16
            ngrp = (cnt + (GC - 1)) // GC

            def grp(t, _):
                o = t * GC
                pltpu.async_copy(msg_h.at[eidb.at[pl.ds(o, GC)]], rowsb,
                                 sem).wait()
                lix = lidb[pl.ds(o, GC)]

                def chan(c_, _):
                    cc = jnp.full((16,), c_, jnp.int32)
                    v = plsc.load_gather(rowsb, [ii16, cc])
                    plsc.addupdate_scatter(acc, [lix, cc], v)
                    return 0

                lax.fori_loop(0, C, chan, 0)
                return 0

            lax.fori_loop(0, ngrp, grp, 0)
            return 0

        lax.fori_loop(0, NR, rnd, 0)
        pltpu.sync_copy(acc.at[pl.ds(0, NPT)], agg_h.at[pl.ds(nodebase, NPT)])

    return k(msg, dst)


def kernel(x, edge_index, batch, params):
    p = params
    src = edge_index[0]
    dst = edge_index[1]

    h = _mm(x, p['W1'], p['b1'])
    for cn, nn in (('c1', 'n1'), ('c2', 'n2'), ('c3', 'n3')):
        xl = _mm(h, p[cn + '_Wl'])
        xr = _mm(h, p[cn + '_Wr'])
        ex = _p1(xl, xr, src, dst, p[cn + '_att'].reshape(HC))
        dfull = _p1b(ex, dst)
        msg = _p2(xl, ex, dfull, src, dst)
        agg = _p3(msg, dst)
        t, ps = _post_a(h, agg, p[cn + '_b'])
        h = _post_b(t, ps, p[nn + '_w'], p[nn + '_b'], p[nn + '_ms'])

    bn = 1.0 / jnp.sqrt(1.0 + 1e-5)
    hout, preds = _head(h, p['e_W1'], p['e_b1'], p['e_g'] * bn, p['e_beta'],
                        p['e_W2'], p['e_b2'], p['p_g'] * bn, p['p_beta'],
                        p['p_W'], p['p_b'])
    return hout, preds


# speedup vs baseline: 4.4860x; 1.4025x over previous
import functools

import jax
import jax.numpy as jnp
from jax import lax
from jax.experimental import pallas as pl
from jax.experimental.pallas import tpu as pltpu
from jax.experimental.pallas import tpu_sc as plsc

F32 = jnp.float32
N = 10000
E = 160000
H = 4
C = 256
HC = H * C

NC, NS = 2, 16
NW = NC * NS
EPW = E // NW
EPT3 = E // NS
B1 = 40
B2 = 40
B3 = 80
NPT = 320
NG = NW * NPT
DB = 2000
NR = E // DB
GC = 16


def _mm(x, w, b=None, act=None, rows=400):
    n, k = x.shape
    m = w.shape[1]
    bb = jnp.zeros((1, m), F32) if b is None else b.reshape(1, m)

    def body(x_ref, w_ref, b_ref, o_ref):
        acc = jnp.dot(x_ref[...], w_ref[...], preferred_element_type=F32,
                      precision=lax.Precision.HIGHEST)
        acc = acc + b_ref[...]
        if act is not None:
            acc = act(acc)
        o_ref[...] = acc

    return pl.pallas_call(
        body,
        grid=(n // rows,),
        in_specs=[pl.BlockSpec((rows, k), lambda i: (i, 0)),
                  pl.BlockSpec((k, m), lambda i: (0, 0)),
                  pl.BlockSpec((1, m), lambda i: (0, 0))],
        out_specs=pl.BlockSpec((rows, m), lambda i: (i, 0)),
        out_shape=jax.ShapeDtypeStruct((n, m), F32),
    )(x, w, bb)


def _post_a(hprev, agg, cb):
    rows = 1000
    g = N // rows

    def body(hp_ref, agg_ref, cb_ref, t_ref, ps_ref):
        t = jnp.maximum(hp_ref[...] + agg_ref[...] * (1.0 / H) + cb_ref[...],
                        0.0)
        t_ref[...] = t
        r8 = t.reshape(rows // 8, 8, C)
        ps_ref[0, 0] = jnp.sum(r8, axis=0)
        ps_ref[0, 1] = jnp.sum(r8 * r8, axis=0)

    return pl.pallas_call(
        body,
        grid=(g,),
        in_specs=[pl.BlockSpec((rows, C), lambda i: (i, 0)),
                  pl.BlockSpec((rows, C), lambda i: (i, 0)),
                  pl.BlockSpec((1, C), lambda i: (0, 0))],
        out_specs=[pl.BlockSpec((rows, C), lambda i: (i, 0)),
                   pl.BlockSpec((1, 2, 8, C), lambda i: (i, 0, 0, 0))],
        out_shape=[jax.ShapeDtypeStruct((N, C), F32),
                   jax.ShapeDtypeStruct((g, 2, 8, C), F32)],
    )(hprev, agg, cb.reshape(1, C))


def _post_b(t, ps, nw, nb, nms):
    rows = 1000
    g = N // rows

    def body(t_ref, ps_ref, w_ref, b_ref, ms_ref, o_ref):
        ps = ps_ref[...]
        mean = jnp.sum(ps[:, 0], axis=(0, 1)) * (1.0 / N)
        msq = jnp.sum(ps[:, 1], axis=(0, 1)) * (1.0 / N)
        mm = mean * ms_ref[0]
        var = msq - mm * (2.0 * mean - mm)
        tt = t_ref[...]
        o_ref[...] = (tt - mm) * lax.rsqrt(var + 1e-5) * w_ref[0] + b_ref[0]

    return pl.pallas_call(
        body,
        grid=(g,),
        in_specs=[pl.BlockSpec((rows, C), lambda i: (i, 0)),
                  pl.BlockSpec((g, 2, 8, C), lambda i: (0, 0, 0, 0)),
                  pl.BlockSpec((1, C), lambda i: (0, 0)),
                  pl.BlockSpec((1, C), lambda i: (0, 0)),
                  pl.BlockSpec((1, C), lambda i: (0, 0))],
        out_specs=pl.BlockSpec((rows, C), lambda i: (i, 0)),
        out_shape=jax.ShapeDtypeStruct((N, C), F32),
    )(t, ps, nw.reshape(1, C), nb.reshape(1, C), nms.reshape(1, C))


def _head(h, w1, b1, s1, t1, w2, b2, s2, t2, pw, pb):
    rows = 400
    g = N // rows

    def body(h_ref, w1_ref, b1_ref, s1_ref, t1_ref, w2_ref, b2_ref,
             s2_ref, t2_ref, pw_ref, pb_ref, ho_ref, po_ref):
        a = jnp.dot(h_ref[...], w1_ref[...], preferred_element_type=F32,
                    precision=lax.Precision.HIGHEST) + b1_ref[...]
        a = jnp.maximum(a, 0.0)
        a = a * s1_ref[...] + t1_ref[...]
        h2 = jnp.dot(a, w2_ref[...], preferred_element_type=F32,
                     precision=lax.Precision.HIGHEST) + b2_ref[...]
        ho_ref[...] = h2
        pin = h2 * s2_ref[...] + t2_ref[...]
        po_ref[...] = jnp.dot(pin, pw_ref[...], preferred_element_type=F32,
                              precision=lax.Precision.HIGHEST) + pb_ref[...]

    vec = lambda v: v.reshape(1, -1)
    return pl.pallas_call(
        body,
        grid=(g,),
        in_specs=[pl.BlockSpec((rows, C), lambda i: (i, 0)),
                  pl.BlockSpec((C, C), lambda i: (0, 0)),
                  pl.BlockSpec((1, C), lambda i: (0, 0)),
                  pl.BlockSpec((1, C), lambda i: (0, 0)),
                  pl.BlockSpec((1, C), lambda i: (0, 0)),
                  pl.BlockSpec((C, C), lambda i: (0, 0)),
                  pl.BlockSpec((1, C), lambda i: (0, 0)),
                  pl.BlockSpec((1, C), lambda i: (0, 0)),
                  pl.BlockSpec((1, C), lambda i: (0, 0)),
                  pl.BlockSpec((C, 128), lambda i: (0, 0)),
                  pl.BlockSpec((1, 128), lambda i: (0, 0))],
        out_specs=[pl.BlockSpec((rows, C), lambda i: (i, 0)),
                   pl.BlockSpec((rows, 128), lambda i: (i, 0))],
        out_shape=[jax.ShapeDtypeStruct((N, C), F32),
                   jax.ShapeDtypeStruct((N, 128), F32)],
    )(h, w1, vec(b1), vec(s1), vec(t1), w2, vec(b2), vec(s2), vec(t2),
      pw, vec(pb))


def _sc_mesh():
    return plsc.VectorSubcoreMesh(core_axis_name="c", subcore_axis_name="s")


def _permute(v, idx):
    return lax.gather(
        v, idx[:, None],
        lax.GatherDimensionNumbers(offset_dims=(), collapsed_slice_dims=(0,),
                                   start_index_map=(0,)),
        (1,), mode=lax.GatherScatterMode.PROMISE_IN_BOUNDS)


def _gather_accumulate(rows_h, eidb, lidb, rowsb, acc, sem, ngrp, aw):
    def start(t):
        return pltpu.async_copy(
            rows_h.at[eidb.at[pl.ds(t * GC, GC)]],
            rowsb.at[pl.ds((t % 2) * GC, GC)], sem)

    @pl.when(ngrp > 0)
    def _():
        start(0)

    def grp(t, _):
        cur = (t % 2) * GC
        pltpu.make_async_copy(rows_h.at[eidb.at[pl.ds(t * GC, GC)]],
                              rowsb.at[pl.ds(cur, GC)], sem).wait()

        @pl.when(t + 1 < ngrp)
        def _():
            start(t + 1)

        lix = lidb[pl.ds(t * GC, GC)]
        for row in range(GC):
            lr = lix[row]
            for ck in range(aw // 16):
                sl = pl.ds(ck * 16, 16)
                acc[lr, sl] = acc[lr, sl] + rowsb[cur + row, sl]
        return 0

    lax.fori_loop(0, ngrp, grp, 0)


def _compact_chunk(dstb, eidb, lidb, kk, r, nodebase, cnt, ii16):
    dv = dstb[pl.ds(kk * 16, 16)]
    loc = dv - nodebase
    m = (loc >= 0) & (loc < NPT)
    eid = r * DB + kk * 16 + ii16
    mi = jnp.where(m, 1, 0).astype(jnp.int32)
    s = mi
    for d in (1, 2, 4, 8):
        sh = _permute(s, jnp.maximum(ii16 - d, 0))
        s = s + jnp.where(ii16 >= d, sh, 0)
    tgt = jnp.where(m, cnt + s - mi, DB + GC)
    plsc.store_scatter(eidb, [tgt], eid)
    plsc.store_scatter(lidb, [tgt], loc)
    return cnt + s[15]


def _p1(xl, xr, src, dst, attf):
    @functools.partial(
        pl.kernel,
        out_type=jax.ShapeDtypeStruct((E, 128), F32),
        mesh=_sc_mesh(),
        compiler_params=pltpu.CompilerParams(needs_layout_passes=False),
        scratch_types=[pltpu.VMEM((B1,), jnp.int32),
                       pltpu.VMEM((B1,), jnp.int32),
                       pltpu.VMEM((B1, HC), F32),
                       pltpu.VMEM((B1, HC), F32),
                       pltpu.VMEM((B1, 128), F32),
                       pltpu.VMEM((HC,), F32),
                       pltpu.SemaphoreType.DMA,
                       pltpu.SemaphoreType.DMA],
    )
    def k(xl_h, xr_h, src_h, dst_h, att_h, ex_h,
          isrc, idst, rl, rr, exb, attv, sem1, sem2):
        cid = lax.axis_index("c")
        sid = lax.axis_index("s")
        wid = sid * NC + cid

        pltpu.sync_copy(att_h, attv)
        z16 = jnp.zeros((16,), F32)

        def zrow(i, _):
            for kk in range(1, 8):
                exb[i, pl.ds(kk * 16, 16)] = z16
            return 0

        lax.fori_loop(0, B1, zrow, 0)
        base = wid * EPW
        ii16 = lax.broadcasted_iota(jnp.int32, (16,), 0)

        def batch(j, _):
            off = base + j * B1
            pltpu.sync_copy(src_h.at[pl.ds(off, B1)], isrc)
            pltpu.sync_copy(dst_h.at[pl.ds(off, B1)], idst)
            cp1 = pltpu.async_copy(xl_h.at[isrc], rl, sem1)
            cp2 = pltpu.async_copy(xr_h.at[idst], rr, sem2)
            cp1.wait()
            cp2.wait()

            def edge(i, _):
                lv = jnp.full((16,), -1e30, F32)
                for h in range(H):
                    a = z16
                    for kk in range(C // 16):
                        o = h * C + kk * 16
                        u = rl[i, pl.ds(o, 16)] + rr[i, pl.ds(o, 16)]
                        a = a + jnp.maximum(u, 0.2 * u) * attv[pl.ds(o, 16)]
                    sh = a[0]
                    for l in range(1, 16):
                        sh = sh + a[l]
                    lv = jnp.where(ii16 == h, sh, lv)
                exb[i, pl.ds(0, 16)] = jnp.exp(lv)
                return 0

            lax.fori_loop(0, B1, edge, 0)
            pltpu.sync_copy(exb, ex_h.at[pl.ds(off, B1)])
            return 0

        lax.fori_loop(0, EPW // B1, batch, 0)

    return k(xl, xr, src, dst, attf)


def _p1b(ex, dst):
    @functools.partial(
        pl.kernel,
        out_type=jax.ShapeDtypeStruct((NG, 128), F32),
        mesh=_sc_mesh(),
        compiler_params=pltpu.CompilerParams(needs_layout_passes=False),
        scratch_types=[pltpu.VMEM((DB,), jnp.int32),
                       pltpu.VMEM((DB + 48,), jnp.int32),
                       pltpu.VMEM((DB + 48,), jnp.int32),
                       pltpu.VMEM((2 * GC, 128), F32),
                       pltpu.VMEM((NPT + 8, 16), F32),
                       pltpu.VMEM((NPT, 128), F32),
                       pltpu.SemaphoreType.DMA],
    )
    def k(ex_h, dst_h, dfull_h, dstb, eidb, lidb, rowsb, acc, stage, sem):
        cid = lax.axis_index("c")
        sid = lax.axis_index("s")
        g = sid * NC + cid
        nodebase = g * NPT

        z16 = jnp.zeros((16,), F32)
        zi16 = jnp.zeros((16,), jnp.int32)
        ti16 = jnp.full((16,), NPT, jnp.int32)
        ii16 = lax.broadcasted_iota(jnp.int32, (16,), 0)

        def zacc(r, _):
            acc[r, :] = z16
            return 0

        lax.fori_loop(0, NPT + 8, zacc, 0)

        def zstage(r, _):
            for kk in range(8):
                stage[r, pl.ds(kk * 16, 16)] = z16
            return 0

        lax.fori_loop(0, NPT, zstage, 0)

        def rnd(r, _):
            pltpu.sync_copy(dst_h.at[pl.ds(r * DB, DB)], dstb)
            cnt = jnp.int32(0)
            for kk in range(DB // 16):
                cnt = _compact_chunk(dstb, eidb, lidb, kk, r, nodebase,
                                     cnt, ii16)
            eidb[pl.ds(cnt, 16)] = zi16
            lidb[pl.ds(cnt, 16)] = ti16
            ngrp = (cnt + (GC - 1)) // GC
            _gather_accumulate(ex_h, eidb, lidb, rowsb, acc, sem, ngrp, 16)
            return 0

        lax.fori_loop(0, NR, rnd, 0)

        def srow(i, _):
            stage[i, pl.ds(0, 16)] = acc[i, :]
            return 0

        lax.fori_loop(0, NPT, srow, 0)
        pltpu.sync_copy(stage, dfull_h.at[pl.ds(nodebase, NPT)])

    return k(ex, dst)


def _p2(xl, ex, dfull, src, dst):
    @functools.partial(
        pl.kernel,
        out_type=jax.ShapeDtypeStruct((E, C), F32),
        mesh=_sc_mesh(),
        compiler_params=pltpu.CompilerParams(needs_layout_passes=False),
        scratch_types=[pltpu.VMEM((B2,), jnp.int32),
                       pltpu.VMEM((B2,), jnp.int32),
                       pltpu.VMEM((B2, HC), F32),
                       pltpu.VMEM((B2, 128), F32),
                       pltpu.VMEM((B2, 128), F32),
                       pltpu.VMEM((B2, C), F32),
                       pltpu.SemaphoreType.DMA,
                       pltpu.SemaphoreType.DMA],
    )
    def k(xl_h, ex_h, dfull_h, src_h, dst_h, msg_h,
          isrc, idst, rl, exb, db, msgb, sem1, sem2):
        cid = lax.axis_index("c")
        sid = lax.axis_index("s")
        wid = sid * NC + cid
        base = wid * EPW

        def batch(j, _):
            off = base + j * B2
            pltpu.sync_copy(src_h.at[pl.ds(off, B2)], isrc)
            pltpu.sync_copy(dst_h.at[pl.ds(off, B2)], idst)
            cp1 = pltpu.async_copy(xl_h.at[isrc], rl, sem1)
            cp2 = pltpu.async_copy(dfull_h.at[idst], db, sem2)
            pltpu.sync_copy(ex_h.at[pl.ds(off, B2)], exb)
            cp1.wait()
            cp2.wait()

            def edge(i, _):
                av = exb[i, pl.ds(0, 16)] / (db[i, pl.ds(0, 16)] + 1e-16)
                aa = [lax.squeeze(lax.slice(av, (h,), (h + 1,)), (0,))
                      for h in range(H)]
                for kk in range(C // 16):
                    mv = aa[0] * rl[i, pl.ds(kk * 16, 16)]
                    mv = mv + aa[1] * rl[i, pl.ds(C + kk * 16, 16)]
                    mv = mv + aa[2] * rl[i, pl.ds(2 * C + kk * 16, 16)]
                    mv = mv + aa[3] * rl[i, pl.ds(3 * C + kk * 16, 16)]
                    msgb[i, pl.ds(kk * 16, 16)] = mv
                return 0

            lax.fori_loop(0, B2, edge, 0)
            pltpu.sync_copy(msgb, msg_h.at[pl.ds(off, B2)])
            return 0

        lax.fori_loop(0, EPW // B2, batch, 0)

    return k(xl, ex, dfull, src, dst)


def _p3(msg, dst):
    @functools.partial(
        pl.kernel,
        out_type=jax.ShapeDtypeStruct((NG, C), F32),
        mesh=_sc_mesh(),
        compiler_params=pltpu.CompilerParams(needs_layout_passes=False),
        scratch_types=[pltpu.VMEM((DB,), jnp.int32),
                       pltpu.VMEM((DB + 48,), jnp.int32),
                       pltpu.VMEM((DB + 48,), jnp.int32),
                       pltpu.VMEM((2 * GC, C), F32),
                       pltpu.VMEM((NPT + 8, C), F32),
                       pltpu.SemaphoreType.DMA],
    )
    def k(msg_h, dst_h, agg_h, dstb, eidb, lidb, rowsb, acc, sem):
        cid = lax.axis_index("c")
        sid = lax.axis_index("s")
        g = sid * NC + cid
        nodebase = g * NPT

        z16 = jnp.zeros((16,), F32)
        zi16 = jnp.zeros((16,), jnp.int32)
        ti16 = jnp.full((16,), NPT, jnp.int32)
        ii16 = lax.broadcasted_iota(jnp.int32, (16,), 0)

        def zacc(r, _):
            for kk in range(C // 16):
                acc[r, pl.ds(kk * 16, 16)] = z16
            return 0

        lax.fori_loop(0, NPT + 8, zacc, 0)

        def rnd(r, _):
            pltpu.sync_copy(dst_h.at[pl.ds(r * DB, DB)], dstb)
            cnt = jnp.int32(0)
            for kk in range(DB // 16):
                cnt = _compact_chunk(dstb, eidb, lidb, kk, r, nodebase,
                                     cnt, ii16)
            eidb[pl.ds(cnt, 16)] = zi16
            lidb[pl.ds(cnt, 16)] = ti16
            ngrp = (cnt + (GC - 1)) // GC
            _gather_accumulate(msg_h, eidb, lidb, rowsb, acc, sem, ngrp, C)
            return 0

        lax.fori_loop(0, NR, rnd, 0)
        pltpu.sync_copy(acc.at[pl.ds(0, NPT)], agg_h.at[pl.ds(nodebase, NPT)])

    return k(msg, dst)


def kernel(x, edge_index, batch, params):
    p = params
    src = edge_index[0]
    dst = edge_index[1]

    h = _mm(x, p['W1'], p['b1'])
    for cn, nn in (('c1', 'n1'), ('c2', 'n2'), ('c3', 'n3')):
        xl = _mm(h, p[cn + '_Wl'])
        xr = _mm(h, p[cn + '_Wr'])
        ex = _p1(xl, xr, src, dst, p[cn + '_att'].reshape(HC))
        dfull = _p1b(ex, dst)
        msg = _p2(xl, ex, dfull, src, dst)
        agg = _p3(msg, dst)
        t, ps = _post_a(h, agg, p[cn + '_b'])
        h = _post_b(t, ps, p[nn + '_w'], p[nn + '_b'], p[nn + '_ms'])

    bn = 1.0 / jnp.sqrt(1.0 + 1e-5)
    hout, preds = _head(h, p['e_W1'], p['e_b1'], p['e_g'] * bn, p['e_beta'],
                        p['e_W2'], p['e_b2'], p['p_g'] * bn, p['p_beta'],
                        p['p_W'], p['p_b'])
    return hout, preds


# speedup vs baseline: 4.6044x; 1.0264x over previous
import functools

import jax
import jax.numpy as jnp
from jax import lax
from jax.experimental import pallas as pl
from jax.experimental.pallas import tpu as pltpu
from jax.experimental.pallas import tpu_sc as plsc

F32 = jnp.float32
N = 10000
E = 160000
H = 4
C = 256
HC = H * C

NC, NS = 2, 16
NW = NC * NS
EPW = E // NW
EPT3 = E // NS
B1 = 40
B2 = 40
B3 = 80
NPT = 320
NG = NW * NPT
DB = 2000
NR = E // DB
GC = 16


def _mm(x, w, b=None, act=None, rows=400):
    n, k = x.shape
    m = w.shape[1]
    bb = jnp.zeros((1, m), F32) if b is None else b.reshape(1, m)

    def body(x_ref, w_ref, b_ref, o_ref):
        acc = jnp.dot(x_ref[...], w_ref[...], preferred_element_type=F32,
                      precision=lax.Precision.HIGHEST)
        acc = acc + b_ref[...]
        if act is not None:
            acc = act(acc)
        o_ref[...] = acc

    return pl.pallas_call(
        body,
        grid=(n // rows,),
        in_specs=[pl.BlockSpec((rows, k), lambda i: (i, 0)),
                  pl.BlockSpec((k, m), lambda i: (0, 0)),
                  pl.BlockSpec((1, m), lambda i: (0, 0))],
        out_specs=pl.BlockSpec((rows, m), lambda i: (i, 0)),
        out_shape=jax.ShapeDtypeStruct((n, m), F32),
    )(x, w, bb)


def _post_a(hprev, agg, cb):
    rows = 1000
    g = N // rows

    def body(hp_ref, agg_ref, cb_ref, t_ref, ps_ref):
        t = jnp.maximum(hp_ref[...] + agg_ref[...] * (1.0 / H) + cb_ref[...],
                        0.0)
        t_ref[...] = t
        r8 = t.reshape(rows // 8, 8, C)
        ps_ref[0, 0] = jnp.sum(r8, axis=0)
        ps_ref[0, 1] = jnp.sum(r8 * r8, axis=0)

    return pl.pallas_call(
        body,
        grid=(g,),
        in_specs=[pl.BlockSpec((rows, C), lambda i: (i, 0)),
                  pl.BlockSpec((rows, C), lambda i: (i, 0)),
                  pl.BlockSpec((1, C), lambda i: (0, 0))],
        out_specs=[pl.BlockSpec((rows, C), lambda i: (i, 0)),
                   pl.BlockSpec((1, 2, 8, C), lambda i: (i, 0, 0, 0))],
        out_shape=[jax.ShapeDtypeStruct((N, C), F32),
                   jax.ShapeDtypeStruct((g, 2, 8, C), F32)],
    )(hprev, agg, cb.reshape(1, C))


def _post_b(t, ps, nw, nb, nms):
    rows = 1000
    g = N // rows

    def body(t_ref, ps_ref, w_ref, b_ref, ms_ref, o_ref):
        ps = ps_ref[...]
        mean = jnp.sum(ps[:, 0], axis=(0, 1)) * (1.0 / N)
        msq = jnp.sum(ps[:, 1], axis=(0, 1)) * (1.0 / N)
        mm = mean * ms_ref[0]
        var = msq - mm * (2.0 * mean - mm)
        tt = t_ref[...]
        o_ref[...] = (tt - mm) * lax.rsqrt(var + 1e-5) * w_ref[0] + b_ref[0]

    return pl.pallas_call(
        body,
        grid=(g,),
        in_specs=[pl.BlockSpec((rows, C), lambda i: (i, 0)),
                  pl.BlockSpec((g, 2, 8, C), lambda i: (0, 0, 0, 0)),
                  pl.BlockSpec((1, C), lambda i: (0, 0)),
                  pl.BlockSpec((1, C), lambda i: (0, 0)),
                  pl.BlockSpec((1, C), lambda i: (0, 0))],
        out_specs=pl.BlockSpec((rows, C), lambda i: (i, 0)),
        out_shape=jax.ShapeDtypeStruct((N, C), F32),
    )(t, ps, nw.reshape(1, C), nb.reshape(1, C), nms.reshape(1, C))


def _head(h, w1, b1, s1, t1, w2, b2, s2, t2, pw, pb):
    rows = 400
    g = N // rows

    def body(h_ref, w1_ref, b1_ref, s1_ref, t1_ref, w2_ref, b2_ref,
             s2_ref, t2_ref, pw_ref, pb_ref, ho_ref, po_ref):
        a = jnp.dot(h_ref[...], w1_ref[...], preferred_element_type=F32,
                    precision=lax.Precision.HIGHEST) + b1_ref[...]
        a = jnp.maximum(a, 0.0)
        a = a * s1_ref[...] + t1_ref[...]
        h2 = jnp.dot(a, w2_ref[...], preferred_element_type=F32,
                     precision=lax.Precision.HIGHEST) + b2_ref[...]
        ho_ref[...] = h2
        pin = h2 * s2_ref[...] + t2_ref[...]
        po_ref[...] = jnp.dot(pin, pw_ref[...], preferred_element_type=F32,
                              precision=lax.Precision.HIGHEST) + pb_ref[...]

    vec = lambda v: v.reshape(1, -1)
    return pl.pallas_call(
        body,
        grid=(g,),
        in_specs=[pl.BlockSpec((rows, C), lambda i: (i, 0)),
                  pl.BlockSpec((C, C), lambda i: (0, 0)),
                  pl.BlockSpec((1, C), lambda i: (0, 0)),
                  pl.BlockSpec((1, C), lambda i: (0, 0)),
                  pl.BlockSpec((1, C), lambda i: (0, 0)),
                  pl.BlockSpec((C, C), lambda i: (0, 0)),
                  pl.BlockSpec((1, C), lambda i: (0, 0)),
                  pl.BlockSpec((1, C), lambda i: (0, 0)),
                  pl.BlockSpec((1, C), lambda i: (0, 0)),
                  pl.BlockSpec((C, 128), lambda i: (0, 0)),
                  pl.BlockSpec((1, 128), lambda i: (0, 0))],
        out_specs=[pl.BlockSpec((rows, C), lambda i: (i, 0)),
                   pl.BlockSpec((rows, 128), lambda i: (i, 0))],
        out_shape=[jax.ShapeDtypeStruct((N, C), F32),
                   jax.ShapeDtypeStruct((N, 128), F32)],
    )(h, w1, vec(b1), vec(s1), vec(t1), w2, vec(b2), vec(s2), vec(t2),
      pw, vec(pb))


def _sc_mesh():
    return plsc.VectorSubcoreMesh(core_axis_name="c", subcore_axis_name="s")


def _permute(v, idx):
    return lax.gather(
        v, idx[:, None],
        lax.GatherDimensionNumbers(offset_dims=(), collapsed_slice_dims=(0,),
                                   start_index_map=(0,)),
        (1,), mode=lax.GatherScatterMode.PROMISE_IN_BOUNDS)


def _gather_accumulate(rows_h, eidb, lidb, rowsb, acc, sem, ngrp, aw):
    def start(t):
        return pltpu.async_copy(
            rows_h.at[eidb.at[pl.ds(t * GC, GC)]],
            rowsb.at[pl.ds((t % 2) * GC, GC)], sem)

    @pl.when(ngrp > 0)
    def _():
        start(0)

    def grp(t, _):
        cur = (t % 2) * GC
        pltpu.make_async_copy(rows_h.at[eidb.at[pl.ds(t * GC, GC)]],
                              rowsb.at[pl.ds(cur, GC)], sem).wait()

        @pl.when(t + 1 < ngrp)
        def _():
            start(t + 1)

        lix = lidb[pl.ds(t * GC, GC)]
        for row in range(GC):
            lr = lix[row]
            for ck in range(aw // 16):
                sl = pl.ds(ck * 16, 16)
                acc[lr, sl] = acc[lr, sl] + rowsb[cur + row, sl]
        return 0

    lax.fori_loop(0, ngrp, grp, 0)


def _compact_chunk(dstb, eidb, lidb, kk, r, nodebase, cnt, ii16):
    dv = dstb[pl.ds(kk * 16, 16)]
    loc = dv - nodebase
    m = (loc >= 0) & (loc < NPT)
    eid = r * DB + kk * 16 + ii16
    mi = jnp.where(m, 1, 0).astype(jnp.int32)
    s = mi
    for d in (1, 2, 4, 8):
        sh = _permute(s, jnp.maximum(ii16 - d, 0))
        s = s + jnp.where(ii16 >= d, sh, 0)
    tgt = jnp.where(m, cnt + s - mi, DB + GC)
    plsc.store_scatter(eidb, [tgt], eid)
    plsc.store_scatter(lidb, [tgt], loc)
    return cnt + s[15]


def _p1(xl, xr, src, dst, attf):
    @functools.partial(
        pl.kernel,
        out_type=jax.ShapeDtypeStruct((E, 128), F32),
        mesh=_sc_mesh(),
        compiler_params=pltpu.CompilerParams(needs_layout_passes=False),
        scratch_types=[pltpu.VMEM((B1,), jnp.int32),
                       pltpu.VMEM((B1,), jnp.int32),
                       pltpu.VMEM((B1, HC), F32),
                       pltpu.VMEM((B1, HC), F32),
                       pltpu.VMEM((B1, 128), F32),
                       pltpu.VMEM((HC,), F32),
                       pltpu.SemaphoreType.DMA,
                       pltpu.SemaphoreType.DMA],
    )
    def k(xl_h, xr_h, src_h, dst_h, att_h, ex_h,
          isrc, idst, rl, rr, exb, attv, sem1, sem2):
        cid = lax.axis_index("c")
        sid = lax.axis_index("s")
        wid = sid * NC + cid

        pltpu.sync_copy(att_h, attv)
        z16 = jnp.zeros((16,), F32)

        def zrow(i, _):
            for kk in range(1, 8):
                exb[i, pl.ds(kk * 16, 16)] = z16
            return 0

        lax.fori_loop(0, B1, zrow, 0)
        base = wid * EPW
        ii16 = lax.broadcasted_iota(jnp.int32, (16,), 0)

        def batch(j, _):
            off = base + j * B1
            pltpu.sync_copy(src_h.at[pl.ds(off, B1)], isrc)
            pltpu.sync_copy(dst_h.at[pl.ds(off, B1)], idst)
            cp1 = pltpu.async_copy(xl_h.at[isrc], rl, sem1)
            cp2 = pltpu.async_copy(xr_h.at[idst], rr, sem2)
            cp1.wait()
            cp2.wait()

            def edge(i, _):
                lv = jnp.full((16,), -1e30, F32)
                for h in range(H):
                    a = z16
                    b = z16
                    for kk in range(0, C // 16, 2):
                        o = h * C + kk * 16
                        u = rl[i, pl.ds(o, 16)] + rr[i, pl.ds(o, 16)]
                        a = a + jnp.maximum(u, 0.2 * u) * attv[pl.ds(o, 16)]
                        o2 = o + 16
                        u2 = rl[i, pl.ds(o2, 16)] + rr[i, pl.ds(o2, 16)]
                        b = b + jnp.maximum(u2, 0.2 * u2) * attv[pl.ds(o2, 16)]
                    v = a + b
                    for d in (8, 4, 2, 1):
                        v = v + _permute(v, jnp.bitwise_xor(ii16, d))
                    lv = jnp.where(ii16 == h, v[0], lv)
                exb[i, pl.ds(0, 16)] = jnp.exp(lv)
                return 0

            lax.fori_loop(0, B1, edge, 0)
            pltpu.sync_copy(exb, ex_h.at[pl.ds(off, B1)])
            return 0

        lax.fori_loop(0, EPW // B1, batch, 0)

    return k(xl, xr, src, dst, attf)


def _p1b(ex, dst):
    @functools.partial(
        pl.kernel,
        out_type=jax.ShapeDtypeStruct((NG, 128), F32),
        mesh=_sc_mesh(),
        compiler_params=pltpu.CompilerParams(needs_layout_passes=False),
        scratch_types=[pltpu.VMEM((DB,), jnp.int32),
                       pltpu.VMEM((DB + 48,), jnp.int32),
                       pltpu.VMEM((DB + 48,), jnp.int32),
                       pltpu.VMEM((2 * GC, 128), F32),
                       pltpu.VMEM((NPT + 8, 16), F32),
                       pltpu.VMEM((NPT, 128), F32),
                       pltpu.SemaphoreType.DMA],
    )
    def k(ex_h, dst_h, dfull_h, dstb, eidb, lidb, rowsb, acc, stage, sem):
        cid = lax.axis_index("c")
        sid = lax.axis_index("s")
        g = sid * NC + cid
        nodebase = g * NPT

        z16 = jnp.zeros((16,), F32)
        zi16 = jnp.zeros((16,), jnp.int32)
        ti16 = jnp.full((16,), NPT, jnp.int32)
        ii16 = lax.broadcasted_iota(jnp.int32, (16,), 0)

        def zacc(r, _):
            acc[r, :] = z16
            return 0

        lax.fori_loop(0, NPT + 8, zacc, 0)

        def zstage(r, _):
            for kk in range(8):
                stage[r, pl.ds(kk * 16, 16)] = z16
            return 0

        lax.fori_loop(0, NPT, zstage, 0)

        def rnd(r, _):
            pltpu.sync_copy(dst_h.at[pl.ds(r * DB, DB)], dstb)
            cnt = jnp.int32(0)
            for kk in range(DB // 16):
                cnt = _compact_chunk(dstb, eidb, lidb, kk, r, nodebase,
                                     cnt, ii16)
            eidb[pl.ds(cnt, 16)] = zi16
            lidb[pl.ds(cnt, 16)] = ti16
            ngrp = (cnt + (GC - 1)) // GC
            _gather_accumulate(ex_h, eidb, lidb, rowsb, acc, sem, ngrp, 16)
            return 0

        lax.fori_loop(0, NR, rnd, 0)

        def srow(i, _):
            stage[i, pl.ds(0, 16)] = acc[i, :]
            return 0

        lax.fori_loop(0, NPT, srow, 0)
        pltpu.sync_copy(stage, dfull_h.at[pl.ds(nodebase, NPT)])

    return k(ex, dst)


def _p2(xl, ex, dfull, src, dst):
    @functools.partial(
        pl.kernel,
        out_type=jax.ShapeDtypeStruct((E, C), F32),
        mesh=_sc_mesh(),
        compiler_params=pltpu.CompilerParams(needs_layout_passes=False),
        scratch_types=[pltpu.VMEM((2 * B2,), jnp.int32),
                       pltpu.VMEM((2 * B2,), jnp.int32),
                       pltpu.VMEM((2 * B2, HC), F32),
                       pltpu.VMEM((2 * B2, 128), F32),
                       pltpu.VMEM((2 * B2, 128), F32),
                       pltpu.VMEM((B2, C), F32),
                       pltpu.SemaphoreType.DMA,
                       pltpu.SemaphoreType.DMA,
                       pltpu.SemaphoreType.DMA],
    )
    def k(xl_h, ex_h, dfull_h, src_h, dst_h, msg_h,
          isrc, idst, rl, exb, db, msgb, sem1, sem2, sem3):
        cid = lax.axis_index("c")
        sid = lax.axis_index("s")
        wid = sid * NC + cid
        base = wid * EPW
        nb = EPW // B2

        def load(j):
            off = base + j * B2
            hh = (j % 2) * B2
            pltpu.sync_copy(src_h.at[pl.ds(off, B2)],
                            isrc.at[pl.ds(hh, B2)])
            pltpu.sync_copy(dst_h.at[pl.ds(off, B2)],
                            idst.at[pl.ds(hh, B2)])
            pltpu.async_copy(xl_h.at[isrc.at[pl.ds(hh, B2)]],
                             rl.at[pl.ds(hh, B2)], sem1)
            pltpu.async_copy(dfull_h.at[idst.at[pl.ds(hh, B2)]],
                             db.at[pl.ds(hh, B2)], sem2)
            pltpu.sync_copy(ex_h.at[pl.ds(off, B2)], exb.at[pl.ds(hh, B2)])

        load(0)

        def batch(j, _):
            off = base + j * B2
            hh = (j % 2) * B2
            pltpu.make_async_copy(xl_h.at[isrc.at[pl.ds(hh, B2)]],
                                  rl.at[pl.ds(hh, B2)], sem1).wait()
            pltpu.make_async_copy(dfull_h.at[idst.at[pl.ds(hh, B2)]],
                                  db.at[pl.ds(hh, B2)], sem2).wait()

            @pl.when(j + 1 < nb)
            def _():
                load(j + 1)

            @pl.when(j > 0)
            def _():
                pltpu.make_async_copy(msgb, msg_h.at[pl.ds(off - B2, B2)],
                                      sem3).wait()

            def edge(i, _):
                av = (exb[hh + i, pl.ds(0, 16)]
                      / (db[hh + i, pl.ds(0, 16)] + 1e-16))
                aa = [lax.squeeze(lax.slice(av, (h,), (h + 1,)), (0,))
                      for h in range(H)]
                for kk in range(C // 16):
                    mv = aa[0] * rl[hh + i, pl.ds(kk * 16, 16)]
                    mv = mv + aa[1] * rl[hh + i, pl.ds(C + kk * 16, 16)]
                    mv = mv + aa[2] * rl[hh + i, pl.ds(2 * C + kk * 16, 16)]
                    mv = mv + aa[3] * rl[hh + i, pl.ds(3 * C + kk * 16, 16)]
                    msgb[i, pl.ds(kk * 16, 16)] = mv
                return 0

            lax.fori_loop(0, B2, edge, 0)
            pltpu.async_copy(msgb, msg_h.at[pl.ds(off, B2)], sem3)
            return 0

        lax.fori_loop(0, nb, batch, 0)
        pltpu.make_async_copy(msgb, msg_h.at[pl.ds(base + (nb - 1) * B2, B2)],
                              sem3).wait()

    return k(xl, ex, dfull, src, dst)


def _p3(msg, dst):
    @functools.partial(
        pl.kernel,
        out_type=jax.ShapeDtypeStruct((NG, C), F32),
        mesh=_sc_mesh(),
        compiler_params=pltpu.CompilerParams(needs_layout_passes=False),
        scratch_types=[pltpu.VMEM((DB,), jnp.int32),
                       pltpu.VMEM((DB + 48,), jnp.int32),
                       pltpu.VMEM((DB + 48,), jnp.int32),
                       pltpu.VMEM((2 * GC, C), F32),
                       pltpu.VMEM((NPT + 8, C), F32),
                       pltpu.SemaphoreType.DMA],
    )
    def k(msg_h, dst_h, agg_h, dstb, eidb, lidb, rowsb, acc, sem):
        cid = lax.axis_index("c")
        sid = lax.axis_index("s")
        g = sid * NC + cid
        nodebase = g * NPT

        z16 = jnp.zeros((16,), F32)
        zi16 = jnp.zeros((16,), jnp.int32)
        ti16 = jnp.full((16,), NPT, jnp.int32)
        ii16 = lax.broadcasted_iota(jnp.int32, (16,), 0)

        def zacc(r, _):
            for kk in range(C // 16):
                acc[r, pl.ds(kk * 16, 16)] = z16
            return 0

        lax.fori_loop(0, NPT + 8, zacc, 0)

        def rnd(r, _):
            pltpu.sync_copy(dst_h.at[pl.ds(r * DB, DB)], dstb)
            cnt = jnp.int32(0)
            for kk in range(DB // 16):
                cnt = _compact_chunk(dstb, eidb, lidb, kk, r, nodebase,
                                     cnt, ii16)
            eidb[pl.ds(cnt, 16)] = zi16
            lidb[pl.ds(cnt, 16)] = ti16
            ngrp = (cnt + (GC - 1)) // GC
            _gather_accumulate(msg_h, eidb, lidb, rowsb, acc, sem, ngrp, C)
            return 0

        lax.fori_loop(0, NR, rnd, 0)
        pltpu.sync_copy(acc.at[pl.ds(0, NPT)], agg_h.at[pl.ds(nodebase, NPT)])

    return k(msg, dst)


def kernel(x, edge_index, batch, params):
    p = params
    src = edge_index[0]
    dst = edge_index[1]

    h = _mm(x, p['W1'], p['b1'])
    for cn, nn in (('c1', 'n1'), ('c2', 'n2'), ('c3', 'n3')):
        xl = _mm(h, p[cn + '_Wl'])
        xr = _mm(h, p[cn + '_Wr'])
        ex = _p1(xl, xr, src, dst, p[cn + '_att'].reshape(HC))
        dfull = _p1b(ex, dst)
        msg = _p2(xl, ex, dfull, src, dst)
        agg = _p3(msg, dst)
        t, ps = _post_a(h, agg, p[cn + '_b'])
        h = _post_b(t, ps, p[nn + '_w'], p[nn + '_b'], p[nn + '_ms'])

    bn = 1.0 / jnp.sqrt(1.0 + 1e-5)
    hout, preds = _head(h, p['e_W1'], p['e_b1'], p['e_g'] * bn, p['e_beta'],
                        p['e_W2'], p['e_b2'], p['p_g'] * bn, p['p_beta'],
                        p['p_W'], p['p_b'])
    return hout, preds
